# baseline (speedup 1.0000x reference)
import functools

import jax
import jax.numpy as jnp
from jax import lax
from jax.experimental import pallas as pl
from jax.experimental.pallas import tpu as pltpu
from jax.experimental.pallas import tpu_sc as plsc

N_CLS = 1024
CTX_LEN = 77
DIM = 768
N_CTX = 20
SUFFIX_START = 1 + N_CTX
N_POS = 1 + (CTX_LEN - SUFFIX_START)
COLS = (0,) + tuple(range(SUFFIX_START, CTX_LEN))


@functools.lru_cache(maxsize=1)
def _build_sc_kernel():
    info = plsc.get_sparse_core_info()
    nw = info.num_cores * info.num_subcores
    cpw = N_CLS // nw
    mesh = plsc.VectorSubcoreMesh(core_axis_name="c", subcore_axis_name="s")

    @functools.partial(
        pl.kernel,
        mesh=mesh,
        out_type=jax.ShapeDtypeStruct((CTX_LEN, N_CLS, DIM), jnp.float32),
        scratch_types=[
            pltpu.VMEM((N_POS * cpw,), jnp.int32),
            pltpu.VMEM((cpw, DIM), jnp.float32),
            pltpu.VMEM((cpw, DIM), jnp.float32),
            pltpu.SemaphoreType.DMA,
            pltpu.SemaphoreType.DMA,
            pltpu.SemaphoreType.DMA,
            pltpu.SemaphoreType.DMA,
        ],
    )
    def body(idx_hbm, ctx_hbm, table_hbm, out_hbm,
             idx_v, buf0, buf1, gsem, osem0, osem1, csem):
        wid = lax.axis_index("s") * info.num_cores + lax.axis_index("c")
        base = wid * cpw
        bufs = (buf0, buf1)
        osems = (osem0, osem1)

        ctx_copy = pltpu.async_copy(
            ctx_hbm, out_hbm.at[pl.ds(1, N_CTX), pl.ds(base, cpw)], csem)

        pltpu.sync_copy(idx_hbm.at[pl.ds(wid * (N_POS * cpw), N_POS * cpw)],
                        idx_v)

        for k, t in enumerate(COLS):
            b = k % 2
            if k >= 2:
                pltpu.make_async_copy(bufs[b],
                                      out_hbm.at[COLS[k - 2], pl.ds(base, cpw)],
                                      osems[b]).wait()
            pltpu.async_copy(
                table_hbm.at[idx_v.at[pl.ds(k * cpw, cpw)]],
                bufs[b], gsem).wait()
            pltpu.async_copy(bufs[b], out_hbm.at[t, pl.ds(base, cpw)],
                             osems[b])

        ctx_copy.wait()
        pltpu.make_async_copy(buf0, out_hbm.at[0, pl.ds(base, cpw)],
                              osem0).wait()
        pltpu.make_async_copy(buf1, out_hbm.at[0, pl.ds(base, cpw)],
                              osem1).wait()

    return body


def kernel(tokenized_prompts, token_embedding, ctx, ctx0, ctx1, mode):
    tok = tokenized_prompts
    ctxs = jnp.where(mode == 0, ctx0, ctx1)
    info = plsc.get_sparse_core_info()
    nw = info.num_cores * info.num_subcores
    cpw = N_CLS // nw
    sel = jnp.concatenate([tok[:, :1], tok[:, SUFFIX_START:]], axis=1)
    idx = sel.reshape(nw, cpw, N_POS).transpose(0, 2, 1).reshape(-1)
    ctx_full = jnp.concatenate([ctx, ctxs], axis=0)
    ctx_rep = jnp.broadcast_to(ctx_full[:, None, :], (N_CTX, cpw, DIM))
    pos_major = _build_sc_kernel()(idx, ctx_rep, token_embedding)
    prompts = jnp.transpose(pos_major, (1, 0, 2))
    return (prompts, tokenized_prompts)


# speedup vs baseline: 10.4567x; 10.4567x over previous
import functools

import jax
import jax.numpy as jnp
from jax import lax
from jax.experimental import pallas as pl
from jax.experimental.pallas import tpu as pltpu
from jax.experimental.pallas import tpu_sc as plsc

N_CLS = 1024
CTX_LEN = 77
DIM = 768
N_CTX = 20
SUFFIX_START = 1 + N_CTX
N_POS = 1 + (CTX_LEN - SUFFIX_START)
LANES = 16
REP = 8


@functools.lru_cache(maxsize=1)
def _build_sc_kernel():
    info = plsc.get_sparse_core_info()
    nw = info.num_cores * info.num_subcores
    cpw = N_CLS // nw
    mesh = plsc.VectorSubcoreMesh(core_axis_name="c", subcore_axis_name="s")

    @functools.partial(
        pl.kernel,
        mesh=mesh,
        out_type=jax.ShapeDtypeStruct((CTX_LEN, N_CLS, DIM), jnp.float32),
        scratch_types=[
            pltpu.VMEM((N_POS * cpw,), jnp.int32),
            pltpu.VMEM((N_CTX * DIM,), jnp.float32),
            pltpu.VMEM((cpw, DIM), jnp.float32),
            pltpu.VMEM((cpw, DIM), jnp.float32),
            pltpu.VMEM((REP, DIM), jnp.float32),
            pltpu.SemaphoreType.DMA,
            pltpu.SemaphoreType.DMA,
            pltpu.SemaphoreType.DMA,
            pltpu.SemaphoreType.DMA,
        ],
    )
    def body(idx_hbm, ctx_hbm, table_hbm, out_hbm,
             idx_v, ctx_v, buf0, buf1, rep_v,
             gsem, osem0, osem1, csem):
        wid = lax.axis_index("s") * info.num_cores + lax.axis_index("c")
        base = wid * cpw
        bufs = (buf0, buf1)
        osems = (osem0, osem1)

        pltpu.sync_copy(idx_hbm.at[pl.ds(wid * (N_POS * cpw), N_POS * cpw)],
                        idx_v)
        pltpu.sync_copy(ctx_hbm, ctx_v)

        def drain_out(b):
            pltpu.make_async_copy(bufs[b], out_hbm.at[0, pl.ds(base, cpw)],
                                  osems[b]).wait()

        def drain_ctx():
            for a in range(cpw // REP):
                pltpu.make_async_copy(rep_v, out_hbm.at[0, pl.ds(base, REP)],
                                      csem).wait()

        def gather_issue(k, b):
            return pltpu.async_copy(
                table_hbm.at[idx_v.at[pl.ds(pl.multiple_of(k * cpw, 8), cpw)]],
                bufs[b], gsem)

        def slot(k, b, drain_cond):
            @pl.when(drain_cond)
            def _():
                drain_out(b)

            g = gather_issue(k, b)

            @pl.when((k >= 2) & (k <= N_CTX))
            def _():
                drain_ctx()

            @pl.when(k <= N_CTX)
            def _():
                off = pl.multiple_of((k - 1) * DIM, 8)
                vs = [ctx_v[pl.ds(off + LANES * c, LANES)]
                      for c in range(DIM // LANES)]
                for j in range(REP):
                    for c in range(DIM // LANES):
                        rep_v[j, pl.ds(LANES * c, LANES)] = vs[c]
                for a in range(cpw // REP):
                    pltpu.async_copy(
                        rep_v, out_hbm.at[k, pl.ds(base + REP * a, REP)],
                        csem)

            g.wait()
            pltpu.async_copy(bufs[b], out_hbm.at[k + N_CTX, pl.ds(base, cpw)],
                             osems[b])

        gather_issue(0, 0).wait()
        pltpu.async_copy(buf0, out_hbm.at[0, pl.ds(base, cpw)], osem0)

        @pl.loop(0, (N_POS - 1) // 2)
        def _(p):
            slot(2 * p + 1, 1, p >= 1)
            slot(2 * p + 2, 0, p >= 0)

        drain_ctx()
        drain_out(0)
        drain_out(1)

    return body


def kernel(tokenized_prompts, token_embedding, ctx, ctx0, ctx1, mode):
    tok = tokenized_prompts
    ctxs = jnp.where(mode == 0, ctx0, ctx1)
    info = plsc.get_sparse_core_info()
    nw = info.num_cores * info.num_subcores
    cpw = N_CLS // nw
    sel = jnp.concatenate([tok[:, :1], tok[:, SUFFIX_START:]], axis=1)
    idx = sel.reshape(nw, cpw, N_POS).transpose(0, 2, 1).reshape(-1)
    ctx_full = jnp.concatenate([ctx, ctxs], axis=0).reshape(-1)
    pos_major = _build_sc_kernel()(idx, ctx_full, token_embedding)
    prompts = jnp.transpose(pos_major, (1, 0, 2))
    return (prompts, tokenized_prompts)
